# Initial kernel scaffold; baseline (speedup 1.0000x reference)
import jax
import jax.numpy as jnp
from jax.experimental import pallas as pl


def kernel(x, segment_ids, weight, bias, mean_scale):
    raise NotImplementedError("write your pallas kernel here")


# speedup vs baseline: 7.0199x; 7.0199x over previous
import jax
import jax.numpy as jnp
from jax import lax
from jax.experimental import pallas as pl
from jax.experimental.pallas import tpu as pltpu

N = 100000
D = 128
B = 64
R = 2000
G = N // R


def _stats_body(ids_ref, x_ref, sums_ref, cnt_ref):
    i = pl.program_id(0)
    ids = ids_ref[0]
    x = x_ref[...]
    rhs = jnp.concatenate([x, x * x], axis=1)
    iota = lax.broadcasted_iota(jnp.int32, (B, 1), 0)
    oh = (iota == ids).astype(jnp.float32)
    part = lax.dot_general(oh, rhs, (((1,), (0,)), ((), ())),
                           preferred_element_type=jnp.float32)
    pcnt = jnp.sum(oh, axis=1, keepdims=True)

    @pl.when(i == 0)
    def _():
        sums_ref[...] = jnp.zeros_like(sums_ref)
        cnt_ref[...] = jnp.zeros_like(cnt_ref)

    sums_ref[...] += part
    cnt_ref[...] += jnp.broadcast_to(pcnt, (B, D))


def _apply_body(ids_ref, x_ref, sums_ref, cnt_ref, w_ref, b_ref, ms_ref,
                out_ref, tab_ref):
    i = pl.program_id(0)

    @pl.when(i == 0)
    def _():
        c = jnp.maximum(cnt_ref[...], 1.0)
        s = sums_ref[:, :D]
        q = sums_ref[:, D:]
        mean = s / c
        mm = mean * ms_ref[...]
        segsq = q - 2.0 * mm * s + c * mm * mm
        rstd = lax.rsqrt(segsq / c + 1e-6)
        scale = w_ref[...] * rstd
        shift = b_ref[...] - mm * scale
        tab_ref[...] = jnp.concatenate([scale, shift], axis=1)

    ids = ids_ref[...]
    iota = lax.broadcasted_iota(jnp.int32, (1, B), 1)
    oh = (ids == iota).astype(jnp.float32)
    g = lax.dot_general(oh, tab_ref[...], (((1,), (0,)), ((), ())),
                        preferred_element_type=jnp.float32)
    out_ref[...] = x_ref[...] * g[:, :D] + g[:, D:]


def _graph_norm(x, seg_row, seg_col, weight, bias, mean_scale):
    stats, cnt = pl.pallas_call(
        _stats_body,
        grid=(G,),
        in_specs=[
            pl.BlockSpec((1, 1, R), lambda i: (i, 0, 0)),
            pl.BlockSpec((R, D), lambda i: (i, 0)),
        ],
        out_specs=[
            pl.BlockSpec((B, 2 * D), lambda i: (0, 0)),
            pl.BlockSpec((B, D), lambda i: (0, 0)),
        ],
        out_shape=[
            jax.ShapeDtypeStruct((B, 2 * D), jnp.float32),
            jax.ShapeDtypeStruct((B, D), jnp.float32),
        ],
    )(seg_row, x)

    out = pl.pallas_call(
        _apply_body,
        grid=(G,),
        in_specs=[
            pl.BlockSpec((R, 1), lambda i: (i, 0)),
            pl.BlockSpec((R, D), lambda i: (i, 0)),
            pl.BlockSpec((B, 2 * D), lambda i: (0, 0)),
            pl.BlockSpec((B, D), lambda i: (0, 0)),
            pl.BlockSpec((1, D), lambda i: (0, 0)),
            pl.BlockSpec((1, D), lambda i: (0, 0)),
            pl.BlockSpec((1, D), lambda i: (0, 0)),
        ],
        out_specs=pl.BlockSpec((R, D), lambda i: (i, 0)),
        out_shape=jax.ShapeDtypeStruct((N, D), jnp.float32),
        scratch_shapes=[pltpu.VMEM((B, 2 * D), jnp.float32)],
    )(seg_col, x, stats, cnt, weight, bias, mean_scale)
    return out


def kernel(x, segment_ids, weight, bias, mean_scale):
    seg = segment_ids.astype(jnp.int32)
    seg_row = seg.reshape(G, 1, R)
    seg_col = seg.reshape(N, 1)
    w = weight.reshape(1, D)
    b = bias.reshape(1, D)
    ms = mean_scale.reshape(1, D)
    return _graph_norm(x, seg_row, seg_col, w, b, ms)


# speedup vs baseline: 10.5678x; 1.5054x over previous
import jax
import jax.numpy as jnp
from jax import lax
from jax.experimental import pallas as pl
from jax.experimental.pallas import tpu as pltpu

N = 100000
D = 128
B = 64
R = 2000
G = N // R


def _stats_body(ids_ref, x_ref, sums_ref, cnt_ref):
    i = pl.program_id(0)
    ids = ids_ref[0]
    x = x_ref[...]
    rhs = jnp.concatenate([x, x * x], axis=1)
    iota = lax.broadcasted_iota(jnp.int32, (B, 1), 0)
    oh = (iota == ids).astype(jnp.float32)
    part = lax.dot_general(oh, rhs, (((1,), (0,)), ((), ())),
                           preferred_element_type=jnp.float32)
    pcnt = jnp.sum(oh, axis=1, keepdims=True)

    @pl.when(i == 0)
    def _():
        sums_ref[...] = jnp.zeros_like(sums_ref)
        cnt_ref[...] = jnp.zeros_like(cnt_ref)

    sums_ref[...] += part
    cnt_ref[...] += jnp.broadcast_to(pcnt, (B, D))


def _apply_body(ids_ref, x_ref, sums_ref, cnt_ref, w_ref, b_ref, ms_ref,
                out_ref, tab_ref):
    i = pl.program_id(0)

    @pl.when(i == 0)
    def _():
        c = jnp.maximum(cnt_ref[...], 1.0)
        s = sums_ref[:, :D]
        q = sums_ref[:, D:]
        mean = s / c
        mm = mean * ms_ref[...]
        segsq = q - 2.0 * mm * s + c * mm * mm
        rstd = lax.rsqrt(segsq / c + 1e-6)
        scale = w_ref[...] * rstd
        shift = b_ref[...] - mm * scale
        tab_ref[...] = jnp.concatenate([scale, shift], axis=1)

    ids = ids_ref[0]
    iota = lax.broadcasted_iota(jnp.int32, (B, 1), 0)
    oh = (iota == ids).astype(jnp.float32)
    g = lax.dot_general(oh, tab_ref[...], (((0,), (0,)), ((), ())),
                        preferred_element_type=jnp.float32)
    out_ref[...] = x_ref[...] * g[:, :D] + g[:, D:]


def _graph_norm(x, seg_row, weight, bias, mean_scale):
    stats, cnt = pl.pallas_call(
        _stats_body,
        grid=(G,),
        in_specs=[
            pl.BlockSpec((1, 1, R), lambda i: (i, 0, 0)),
            pl.BlockSpec((R, D), lambda i: (i, 0)),
        ],
        out_specs=[
            pl.BlockSpec((B, 2 * D), lambda i: (0, 0)),
            pl.BlockSpec((B, D), lambda i: (0, 0)),
        ],
        out_shape=[
            jax.ShapeDtypeStruct((B, 2 * D), jnp.float32),
            jax.ShapeDtypeStruct((B, D), jnp.float32),
        ],
    )(seg_row, x)

    out = pl.pallas_call(
        _apply_body,
        grid=(G,),
        in_specs=[
            pl.BlockSpec((1, 1, R), lambda i: (i, 0, 0)),
            pl.BlockSpec((R, D), lambda i: (i, 0)),
            pl.BlockSpec((B, 2 * D), lambda i: (0, 0)),
            pl.BlockSpec((B, D), lambda i: (0, 0)),
            pl.BlockSpec((1, D), lambda i: (0, 0)),
            pl.BlockSpec((1, D), lambda i: (0, 0)),
            pl.BlockSpec((1, D), lambda i: (0, 0)),
        ],
        out_specs=pl.BlockSpec((R, D), lambda i: (i, 0)),
        out_shape=jax.ShapeDtypeStruct((N, D), jnp.float32),
        scratch_shapes=[pltpu.VMEM((B, 2 * D), jnp.float32)],
    )(seg_row, x, stats, cnt, weight, bias, mean_scale)
    return out


def kernel(x, segment_ids, weight, bias, mean_scale):
    seg = segment_ids.astype(jnp.int32)
    seg_row = seg.reshape(G, 1, R)
    w = weight.reshape(1, D)
    b = bias.reshape(1, D)
    ms = mean_scale.reshape(1, D)
    return _graph_norm(x, seg_row, w, b, ms)
